# baseline (speedup 1.0000x reference)
import functools

import jax
import jax.numpy as jnp
from jax import lax
from jax.experimental import pallas as pl
from jax.experimental.pallas import tpu as pltpu
from jax.experimental.pallas import tpu_sc as plsc

N = 10000
E = 320000
D = 128
H = 64
H2 = 2 * H

NC = 2
NS = 16
EPC = E // NC
EPW = E // (NC * NS)
CHUNK = 80
NCHUNK = EPW // CHUNK
NBUF_CNT = 5
NBUF_PLAIN = 7
N_PAD = 10240
ROWS_PW = N_PAD // NS
CW = 16

_f32 = jnp.float32
_sc_mesh = plsc.VectorSubcoreMesh(core_axis_name="c", subcore_axis_name="s")


def _sc_body(with_cnt, nbuf, *refs):
    if with_cnt:
        (src_hbm, dst_hbm, p_hbm, z64_hbm, z16_hbm, ones_hbm,
         out_s, out_c,
         acc, cntacc, src_v, dst_v, ones_v) = refs[:13]
        rows = refs[13:13 + nbuf]
        sems = refs[13 + nbuf:13 + 2 * nbuf]
    else:
        (src_hbm, dst_hbm, p_hbm, z64_hbm,
         out_s,
         acc, src_v, dst_v) = refs[:8]
        rows = refs[8:8 + nbuf]
        sems = refs[8 + nbuf:8 + 2 * nbuf]
    c = lax.axis_index("c")
    s = lax.axis_index("s")
    rbase = s * ROWS_PW
    pltpu.sync_copy(z64_hbm.at[pl.ds(rbase, ROWS_PW)],
                    acc.at[pl.ds(rbase, ROWS_PW)])
    pltpu.sync_copy(src_hbm.at[c, s], src_v)
    pltpu.sync_copy(dst_hbm.at[c, s], dst_v)
    if with_cnt:
        pltpu.sync_copy(z16_hbm.at[pl.ds(rbase, ROWS_PW)],
                        cntacc.at[pl.ds(rbase, ROWS_PW)])
        pltpu.sync_copy(ones_hbm, ones_v)
    plsc.subcore_barrier()

    for b in range(nbuf):
        pltpu.async_copy(p_hbm.at[src_v.at[b]], rows[b], sems[b])

    def _process(j, b, issue_next):
        pltpu.make_async_copy(p_hbm.at[src_v.at[j]], rows[b], sems[b]).wait()
        pltpu.sync_copy(rows[b], acc.at[dst_v.at[j]], add=True)
        if with_cnt:
            pltpu.sync_copy(ones_v, cntacc.at[dst_v.at[j]], add=True)
        if issue_next:
            pltpu.async_copy(p_hbm.at[src_v.at[j + nbuf]], rows[b], sems[b])

    nfull = (NCHUNK // nbuf - 1) * nbuf
    @pl.loop(0, nfull, step=nbuf)
    def _(g):
        for b in range(nbuf):
            _process(g + b, b, True)

    for j in range(nfull, NCHUNK - nbuf):
        _process(j, j % nbuf, True)
    for j in range(NCHUNK - nbuf, NCHUNK):
        _process(j, j % nbuf, False)

    plsc.subcore_barrier()
    pltpu.sync_copy(acc.at[pl.ds(rbase, ROWS_PW)],
                    out_s.at[c, pl.ds(rbase, ROWS_PW), 0])
    if with_cnt:
        pltpu.sync_copy(cntacc.at[pl.ds(rbase, ROWS_PW)],
                        out_c.at[c, pl.ds(rbase, ROWS_PW)])


_sc_seg_sum_cnt = pl.kernel(
    functools.partial(_sc_body, True, NBUF_CNT),
    out_type=[jax.ShapeDtypeStruct((NC, N_PAD, 2, H), _f32),
              jax.ShapeDtypeStruct((NC, N_PAD, CW), _f32)],
    mesh=_sc_mesh,
    compiler_params=pltpu.CompilerParams(use_tc_tiling_on_sc=False),
    scratch_types=(
        [pltpu.VMEM_SHARED((N_PAD, H), _f32),
         pltpu.VMEM_SHARED((N_PAD, CW), _f32),
         pltpu.VMEM((NCHUNK, CHUNK), jnp.int32),
         pltpu.VMEM((NCHUNK, CHUNK), jnp.int32),
         pltpu.VMEM((CHUNK, CW), _f32)]
        + [pltpu.VMEM((CHUNK, H), _f32)] * NBUF_CNT
        + [pltpu.SemaphoreType.DMA] * NBUF_CNT
    ),
)

_sc_seg_sum = pl.kernel(
    functools.partial(_sc_body, False, NBUF_PLAIN),
    out_type=jax.ShapeDtypeStruct((NC, N_PAD, 2, H), _f32),
    mesh=_sc_mesh,
    compiler_params=pltpu.CompilerParams(use_tc_tiling_on_sc=False),
    scratch_types=(
        [pltpu.VMEM_SHARED((N_PAD, H), _f32),
         pltpu.VMEM((NCHUNK, CHUNK), jnp.int32),
         pltpu.VMEM((NCHUNK, CHUNK), jnp.int32)]
        + [pltpu.VMEM((CHUNK, H), _f32)] * NBUF_PLAIN
        + [pltpu.SemaphoreType.DMA] * NBUF_PLAIN
    ),
)


_BLK = 2000
_GRID = N // _BLK


def _pre_body(x_ref, wc_ref, wp_ref, pr_ref, res_ref):
    xb = x_ref[...]
    pr_ref[...] = jnp.dot(xb, wc_ref[...], preferred_element_type=_f32)
    res_ref[...] = jnp.dot(xb, wp_ref[...], preferred_element_type=_f32)


_pre = pl.pallas_call(
    _pre_body,
    grid=(_GRID,),
    in_specs=[
        pl.BlockSpec((_BLK, D), lambda i: (i, 0)),
        pl.BlockSpec((D, H2), lambda i: (0, 0)),
        pl.BlockSpec((D, H), lambda i: (0, 0)),
    ],
    out_specs=[
        pl.BlockSpec((_BLK, H2), lambda i: (i, 0)),
        pl.BlockSpec((_BLK, H), lambda i: (i, 0)),
    ],
    out_shape=[jax.ShapeDtypeStruct((N, H2), _f32),
               jax.ShapeDtypeStruct((N, H), _f32)],
)


def _combine(s_ref, c_ref, pr_ref, bl_ref, g_ref, b_ref, res_ref):
    ssum = s_ref[0, :, 0:H] + s_ref[1, :, 0:H]
    cnt = c_ref[0, :, 0:1] + c_ref[1, :, 0:1]
    agg = ssum / jnp.maximum(cnt, 1.0)
    z = agg + bl_ref[...] + pr_ref[:, H:H2]
    mu = jnp.mean(z, axis=-1, keepdims=True)
    d = z - mu
    var = jnp.mean(d * d, axis=-1, keepdims=True)
    zn = d * lax.rsqrt(var + 1e-5) * g_ref[...] + b_ref[...]
    return jnp.maximum(zn, 0.0) + res_ref[...]


def _post_mid_body(s_ref, c_ref, pr_ref, bl_ref, g_ref, b_ref, res_ref,
                   wc_ref, h_ref, prn_ref):
    h = _combine(s_ref, c_ref, pr_ref, bl_ref, g_ref, b_ref, res_ref)
    h_ref[...] = h
    prn_ref[...] = jnp.dot(h, wc_ref[...], preferred_element_type=_f32)


_post_mid = pl.pallas_call(
    _post_mid_body,
    grid=(_GRID,),
    in_specs=[
        pl.BlockSpec((NC, _BLK, H2), lambda i: (0, i, 0)),
        pl.BlockSpec((NC, _BLK, CW), lambda i: (0, i, 0)),
        pl.BlockSpec((_BLK, H2), lambda i: (i, 0)),
        pl.BlockSpec((1, H), lambda i: (0, 0)),
        pl.BlockSpec((1, H), lambda i: (0, 0)),
        pl.BlockSpec((1, H), lambda i: (0, 0)),
        pl.BlockSpec((_BLK, H), lambda i: (i, 0)),
        pl.BlockSpec((H, H2), lambda i: (0, 0)),
    ],
    out_specs=[
        pl.BlockSpec((_BLK, H), lambda i: (i, 0)),
        pl.BlockSpec((_BLK, H2), lambda i: (i, 0)),
    ],
    out_shape=[jax.ShapeDtypeStruct((N, H), _f32),
               jax.ShapeDtypeStruct((N, H2), _f32)],
)


def _post_last_body(s_ref, c_ref, pr_ref, bl_ref, g_ref, b_ref, res_ref,
                    wh_ref, bh_ref, h_ref, y_ref):
    h = _combine(s_ref, c_ref, pr_ref, bl_ref, g_ref, b_ref, res_ref)
    h_ref[...] = h
    y_ref[...] = jnp.dot(h, wh_ref[...], preferred_element_type=_f32) + bh_ref[...]


_post_last = pl.pallas_call(
    _post_last_body,
    grid=(_GRID,),
    in_specs=[
        pl.BlockSpec((NC, _BLK, H2), lambda i: (0, i, 0)),
        pl.BlockSpec((NC, _BLK, CW), lambda i: (0, i, 0)),
        pl.BlockSpec((_BLK, H2), lambda i: (i, 0)),
        pl.BlockSpec((1, H), lambda i: (0, 0)),
        pl.BlockSpec((1, H), lambda i: (0, 0)),
        pl.BlockSpec((1, H), lambda i: (0, 0)),
        pl.BlockSpec((_BLK, H), lambda i: (i, 0)),
        pl.BlockSpec((H, 1), lambda i: (0, 0)),
        pl.BlockSpec((1, 1), lambda i: (0, 0)),
    ],
    out_specs=[
        pl.BlockSpec((_BLK, H), lambda i: (i, 0)),
        pl.BlockSpec((_BLK, 1), lambda i: (i, 0)),
    ],
    out_shape=[jax.ShapeDtypeStruct((N, H), _f32),
               jax.ShapeDtypeStruct((N, 1), _f32)],
)


def kernel(x, edge_index, Wl0, bl0, Wr0, ln_g0, ln_b0, Wl1, bl1, Wr1,
           ln_g1, ln_b1, Wl2, bl2, Wr2, ln_g2, ln_b2, Wproj, Whead, bhead):
    src = (edge_index[0] * 2).reshape(NC, NS, NCHUNK, CHUNK)
    dst = edge_index[1].reshape(NC, NS, NCHUNK, CHUNK)
    z64 = jnp.zeros((N_PAD, H), _f32)
    z16 = jnp.zeros((N_PAD, CW), _f32)
    ones = jnp.ones((CHUNK, CW), _f32)

    W0c = jnp.concatenate([Wl0.T, Wr0.T], axis=1)
    W1c = jnp.concatenate([Wl1.T, Wr1.T], axis=1)
    W2c = jnp.concatenate([Wl2.T, Wr2.T], axis=1)
    bl0r, g0r, b0r = bl0.reshape(1, H), ln_g0.reshape(1, H), ln_b0.reshape(1, H)
    bl1r, g1r, b1r = bl1.reshape(1, H), ln_g1.reshape(1, H), ln_b1.reshape(1, H)
    bl2r, g2r, b2r = bl2.reshape(1, H), ln_g2.reshape(1, H), ln_b2.reshape(1, H)

    pr0, res0 = _pre(x, W0c, Wproj.T)
    s0, cpart = _sc_seg_sum_cnt(src, dst, pr0.reshape(2 * N, H), z64, z16, ones)
    s0 = s0.reshape(NC, N_PAD, H2)
    h1, pr1 = _post_mid(s0, cpart, pr0, bl0r, g0r, b0r, res0, W1c)
    s1 = _sc_seg_sum(src, dst, pr1.reshape(2 * N, H), z64).reshape(NC, N_PAD, H2)
    h2, pr2 = _post_mid(s1, cpart, pr1, bl1r, g1r, b1r, h1, W2c)
    s2 = _sc_seg_sum(src, dst, pr2.reshape(2 * N, H), z64).reshape(NC, N_PAD, H2)
    h3, y = _post_last(s2, cpart, pr2, bl2r, g2r, b2r, h2,
                       Whead.T, bhead.reshape(1, 1))
    return (y[:, 0], h3)


# speedup vs baseline: 1.6448x; 1.6448x over previous
import functools

import jax
import jax.numpy as jnp
from jax import lax
from jax.experimental import pallas as pl
from jax.experimental.pallas import tpu as pltpu
from jax.experimental.pallas import tpu_sc as plsc

N = 10000
E = 320000
D = 128
H = 64
H2 = 2 * H

NC = 2
NS = 16
EPC = E // NC
EPW = E // (NC * NS)
CHUNK = 80
NCHUNK = EPW // CHUNK
NBUF_CNT = 5
NBUF_PLAIN = 7
N_PAD = 10240
ROWS_PW = N_PAD // NS
CW = 16

_f32 = jnp.float32
_sc_mesh = plsc.VectorSubcoreMesh(core_axis_name="c", subcore_axis_name="s")


def _sc_body(with_cnt, nbuf, *refs):
    if with_cnt:
        (src_hbm, dst_hbm, p_hbm, z64_hbm, z16_hbm, ones_hbm,
         out_s, out_c,
         acc, cntacc, src_v, dst_v, ones_v) = refs[:13]
        rows = refs[13:13 + nbuf]
        sems = refs[13 + nbuf:13 + 2 * nbuf]
    else:
        (src_hbm, dst_hbm, p_hbm, z64_hbm,
         out_s,
         acc, src_v, dst_v) = refs[:8]
        rows = refs[8:8 + nbuf]
        sems = refs[8 + nbuf:8 + 2 * nbuf]
    c = lax.axis_index("c")
    s = lax.axis_index("s")
    rbase = s * ROWS_PW
    pltpu.sync_copy(z64_hbm.at[pl.ds(rbase, ROWS_PW)],
                    acc.at[pl.ds(rbase, ROWS_PW)])
    pltpu.sync_copy(src_hbm.at[c, s], src_v)
    pltpu.sync_copy(dst_hbm.at[c, s], dst_v)
    if with_cnt:
        pltpu.sync_copy(z16_hbm.at[pl.ds(rbase, ROWS_PW)],
                        cntacc.at[pl.ds(rbase, ROWS_PW)])
        pltpu.sync_copy(ones_hbm, ones_v)
    plsc.subcore_barrier()

    for b in range(nbuf):
        pltpu.async_copy(p_hbm.at[src_v.at[b]], rows[b], sems[b])

    def _process(j, b, issue_next):
        pltpu.make_async_copy(p_hbm.at[src_v.at[j]], rows[b], sems[b]).wait()
        pltpu.sync_copy(rows[b], acc.at[dst_v.at[j]], add=True)
        if with_cnt:
            pltpu.sync_copy(ones_v, cntacc.at[dst_v.at[j]], add=True)
        if issue_next:
            pltpu.async_copy(p_hbm.at[src_v.at[j + nbuf]], rows[b], sems[b])

    nfull = (NCHUNK // nbuf - 1) * nbuf
    @pl.loop(0, nfull, step=nbuf)
    def _(g):
        for b in range(nbuf):
            _process(g + b, b, True)

    for j in range(nfull, NCHUNK - nbuf):
        _process(j, j % nbuf, True)
    for j in range(NCHUNK - nbuf, NCHUNK):
        _process(j, j % nbuf, False)

    plsc.subcore_barrier()
    pltpu.sync_copy(acc.at[pl.ds(rbase, ROWS_PW)],
                    out_s.at[c, pl.ds(rbase, ROWS_PW)])
    if with_cnt:
        pltpu.sync_copy(cntacc.at[pl.ds(rbase, ROWS_PW)],
                        out_c.at[c, pl.ds(rbase, ROWS_PW)])


_sc_seg_sum_cnt = pl.kernel(
    functools.partial(_sc_body, True, NBUF_CNT),
    out_type=[jax.ShapeDtypeStruct((NC, N_PAD, H), _f32),
              jax.ShapeDtypeStruct((NC, N_PAD, CW), _f32)],
    mesh=_sc_mesh,
    compiler_params=pltpu.CompilerParams(use_tc_tiling_on_sc=False),
    scratch_types=(
        [pltpu.VMEM_SHARED((N_PAD, H), _f32),
         pltpu.VMEM_SHARED((N_PAD, CW), _f32),
         pltpu.VMEM((NCHUNK, CHUNK), jnp.int32),
         pltpu.VMEM((NCHUNK, CHUNK), jnp.int32),
         pltpu.VMEM((CHUNK, CW), _f32)]
        + [pltpu.VMEM((CHUNK, H), _f32)] * NBUF_CNT
        + [pltpu.SemaphoreType.DMA] * NBUF_CNT
    ),
)

_sc_seg_sum = pl.kernel(
    functools.partial(_sc_body, False, NBUF_PLAIN),
    out_type=jax.ShapeDtypeStruct((NC, N_PAD, H), _f32),
    mesh=_sc_mesh,
    compiler_params=pltpu.CompilerParams(use_tc_tiling_on_sc=False),
    scratch_types=(
        [pltpu.VMEM_SHARED((N_PAD, H), _f32),
         pltpu.VMEM((NCHUNK, CHUNK), jnp.int32),
         pltpu.VMEM((NCHUNK, CHUNK), jnp.int32)]
        + [pltpu.VMEM((CHUNK, H), _f32)] * NBUF_PLAIN
        + [pltpu.SemaphoreType.DMA] * NBUF_PLAIN
    ),
)


_BLK = 2000
_GRID = N // _BLK


def _pre_body(x_ref, wc_ref, wp_ref, pr_ref, res_ref):
    xb = x_ref[...]
    pr_ref[...] = jnp.dot(xb, wc_ref[...], preferred_element_type=_f32)
    res_ref[...] = jnp.dot(xb, wp_ref[...], preferred_element_type=_f32)


_pre = pl.pallas_call(
    _pre_body,
    grid=(_GRID,),
    in_specs=[
        pl.BlockSpec((_BLK, D), lambda i: (i, 0)),
        pl.BlockSpec((D, H2), lambda i: (0, 0)),
        pl.BlockSpec((D, H), lambda i: (0, 0)),
    ],
    out_specs=[
        pl.BlockSpec((_BLK, H2), lambda i: (i, 0)),
        pl.BlockSpec((_BLK, H), lambda i: (i, 0)),
    ],
    out_shape=[jax.ShapeDtypeStruct((N, H2), _f32),
               jax.ShapeDtypeStruct((N, H), _f32)],
)


def _combine(s_ref, c_ref, pr_ref, bl_ref, g_ref, b_ref, res_ref):
    ssum = s_ref[0] + s_ref[1]
    cnt = c_ref[0, :, 0:1] + c_ref[1, :, 0:1]
    agg = ssum / jnp.maximum(cnt, 1.0)
    z = agg + bl_ref[...] + pr_ref[:, H:H2]
    mu = jnp.mean(z, axis=-1, keepdims=True)
    d = z - mu
    var = jnp.mean(d * d, axis=-1, keepdims=True)
    zn = d * lax.rsqrt(var + 1e-5) * g_ref[...] + b_ref[...]
    return jnp.maximum(zn, 0.0) + res_ref[...]


def _post_mid_body(s_ref, c_ref, pr_ref, bl_ref, g_ref, b_ref, res_ref,
                   wc_ref, h_ref, prn_ref):
    h = _combine(s_ref, c_ref, pr_ref, bl_ref, g_ref, b_ref, res_ref)
    h_ref[...] = h
    prn_ref[...] = jnp.dot(h, wc_ref[...], preferred_element_type=_f32)


_post_mid = pl.pallas_call(
    _post_mid_body,
    grid=(_GRID,),
    in_specs=[
        pl.BlockSpec((NC, _BLK, H), lambda i: (0, i, 0)),
        pl.BlockSpec((NC, _BLK, CW), lambda i: (0, i, 0)),
        pl.BlockSpec((_BLK, H2), lambda i: (i, 0)),
        pl.BlockSpec((1, H), lambda i: (0, 0)),
        pl.BlockSpec((1, H), lambda i: (0, 0)),
        pl.BlockSpec((1, H), lambda i: (0, 0)),
        pl.BlockSpec((_BLK, H), lambda i: (i, 0)),
        pl.BlockSpec((H, H2), lambda i: (0, 0)),
    ],
    out_specs=[
        pl.BlockSpec((_BLK, H), lambda i: (i, 0)),
        pl.BlockSpec((_BLK, H2), lambda i: (i, 0)),
    ],
    out_shape=[jax.ShapeDtypeStruct((N, H), _f32),
               jax.ShapeDtypeStruct((N, H2), _f32)],
)


def _post_last_body(s_ref, c_ref, pr_ref, bl_ref, g_ref, b_ref, res_ref,
                    wh_ref, bh_ref, h_ref, y_ref):
    h = _combine(s_ref, c_ref, pr_ref, bl_ref, g_ref, b_ref, res_ref)
    h_ref[...] = h
    y_ref[...] = jnp.dot(h, wh_ref[...], preferred_element_type=_f32) + bh_ref[...]


_post_last = pl.pallas_call(
    _post_last_body,
    grid=(_GRID,),
    in_specs=[
        pl.BlockSpec((NC, _BLK, H), lambda i: (0, i, 0)),
        pl.BlockSpec((NC, _BLK, CW), lambda i: (0, i, 0)),
        pl.BlockSpec((_BLK, H2), lambda i: (i, 0)),
        pl.BlockSpec((1, H), lambda i: (0, 0)),
        pl.BlockSpec((1, H), lambda i: (0, 0)),
        pl.BlockSpec((1, H), lambda i: (0, 0)),
        pl.BlockSpec((_BLK, H), lambda i: (i, 0)),
        pl.BlockSpec((H, 1), lambda i: (0, 0)),
        pl.BlockSpec((1, 1), lambda i: (0, 0)),
    ],
    out_specs=[
        pl.BlockSpec((_BLK, H), lambda i: (i, 0)),
        pl.BlockSpec((_BLK, 1), lambda i: (i, 0)),
    ],
    out_shape=[jax.ShapeDtypeStruct((N, H), _f32),
               jax.ShapeDtypeStruct((N, 1), _f32)],
)


def kernel(x, edge_index, Wl0, bl0, Wr0, ln_g0, ln_b0, Wl1, bl1, Wr1,
           ln_g1, ln_b1, Wl2, bl2, Wr2, ln_g2, ln_b2, Wproj, Whead, bhead):
    src = (edge_index[0] * 2).reshape(NC, NS, NCHUNK, CHUNK)
    dst = edge_index[1].reshape(NC, NS, NCHUNK, CHUNK)
    z64 = jnp.zeros((N_PAD, H), _f32)
    z16 = jnp.zeros((N_PAD, CW), _f32)
    ones = jnp.ones((CHUNK, CW), _f32)

    W0c = jnp.concatenate([Wl0.T, Wr0.T], axis=1)
    W1c = jnp.concatenate([Wl1.T, Wr1.T], axis=1)
    W2c = jnp.concatenate([Wl2.T, Wr2.T], axis=1)
    bl0r, g0r, b0r = bl0.reshape(1, H), ln_g0.reshape(1, H), ln_b0.reshape(1, H)
    bl1r, g1r, b1r = bl1.reshape(1, H), ln_g1.reshape(1, H), ln_b1.reshape(1, H)
    bl2r, g2r, b2r = bl2.reshape(1, H), ln_g2.reshape(1, H), ln_b2.reshape(1, H)

    pr0, res0 = _pre(x, W0c, Wproj.T)
    s0, cpart = _sc_seg_sum_cnt(src, dst, pr0.reshape(2 * N, H), z64, z16, ones)
    h1, pr1 = _post_mid(s0, cpart, pr0, bl0r, g0r, b0r, res0, W1c)
    s1 = _sc_seg_sum(src, dst, pr1.reshape(2 * N, H), z64)
    h2, pr2 = _post_mid(s1, cpart, pr1, bl1r, g1r, b1r, h1, W2c)
    s2 = _sc_seg_sum(src, dst, pr2.reshape(2 * N, H), z64)
    h3, y = _post_last(s2, cpart, pr2, bl2r, g2r, b2r, h2,
                       Whead.T, bhead.reshape(1, 1))
    return (y[:, 0], h3)
